# baseline (speedup 1.0000x reference)
import functools

import jax
import jax.numpy as jnp
from jax import lax
from jax.experimental import pallas as pl
from jax.experimental.pallas import tpu as pltpu
from jax.experimental.pallas import tpu_sc as plsc

_N = 10000
_NP = 10240
_NACC = 10240
_D = 128
_E = 320000
_C = 80
_CPT = 128
_NW = 32
_EPAD = _C * _CPT * _NW
_RPS = _NACC // 16
_NBUF = 4
_NGRP = _CPT // _NBUF


_sc_mesh = plsc.VectorSubcoreMesh(core_axis_name="c", subcore_axis_name="s")


def _make_sc_agg(with_deg):
    scratch = [
        pltpu.VMEM((2 * _NBUF, 128), jnp.int32),
        pltpu.VMEM((2 * _NBUF, 128), jnp.int32),
    ]
    scratch += [pltpu.VMEM((_C, _D), jnp.float32) for _ in range(_NBUF)]
    if with_deg:
        scratch += [pltpu.VMEM((_C,), jnp.float32)]
    scratch += [pltpu.VMEM_SHARED((_NACC, _D), jnp.float32)]
    if with_deg:
        scratch += [pltpu.VMEM_SHARED((_NACC,), jnp.float32)]
    nsem = 2 * _NBUF + 4 + (_NBUF if with_deg else 0)
    scratch += [pltpu.SemaphoreType.DMA for _ in range(nsem)]
    if with_deg:
        out_type = (
            jax.ShapeDtypeStruct((2, _NP, _D), jnp.float32),
            jax.ShapeDtypeStruct((2, _NP), jnp.float32),
        )
    else:
        out_type = jax.ShapeDtypeStruct((2, _NP, _D), jnp.float32)

    def body(h_hbm, src_hbm, dst_hbm, zrow_hbm, *rest):
        if with_deg:
            zvec_hbm, z_out, deg_out, isrc_v, idst_v = rest[:5]
            bufs = rest[5:]
        else:
            z_out, isrc_v, idst_v = rest[:3]
            bufs = rest[3:]
        rows = bufs[:_NBUF]
        k = _NBUF
        if with_deg:
            ones_v = bufs[k]
            k += 1
        z_sh = bufs[k]
        k += 1
        if with_deg:
            deg_sh = bufs[k]
            k += 1
        sems = bufs[k:]
        gsem = sems[:_NBUF]
        ssem = sems[_NBUF:2 * _NBUF]
        off = 2 * _NBUF
        if with_deg:
            dsem = sems[off:off + _NBUF]
            off += _NBUF
        isem = sems[off:off + 2]
        jsem = sems[off + 2:off + 4]

        c = lax.axis_index("c")
        s = lax.axis_index("s")
        base = (c * 16 + s) * _NGRP
        pltpu.sync_copy(zrow_hbm.at[pl.ds(s * _RPS, _RPS)],
                        z_sh.at[pl.ds(s * _RPS, _RPS)])
        if with_deg:
            pltpu.sync_copy(zvec_hbm.at[pl.ds(s * _RPS, _RPS)],
                            deg_sh.at[pl.ds(s * _RPS, _RPS)])
            for i in range(_C // 16):
                ones_v[pl.ds(i * 16, 16)] = jnp.full((16,), 1.0, jnp.float32)
        plsc.subcore_barrier()

        def ifetch(g, slot):
            pltpu.async_copy(src_hbm.at[base + g],
                             isrc_v.at[pl.ds(slot * _NBUF, _NBUF)],
                             isem[slot])
            pltpu.async_copy(dst_hbm.at[base + g],
                             idst_v.at[pl.ds(slot * _NBUF, _NBUF)],
                             jsem[slot])

        def iwait(sem):
            pltpu.make_async_copy(src_hbm.at[0],
                                  isrc_v.at[pl.ds(0, _NBUF)], sem).wait()

        ifetch(0, 0)
        ifetch(1, 1)
        iwait(isem[0])
        for b in range(_NBUF):
            pltpu.async_copy(h_hbm.at[isrc_v.at[b, pl.ds(0, _C)]],
                             rows[b], gsem[b])

        def sub(g, slot):
            nslot = 1 - slot
            iwait(jsem[slot])
            for b in range(_NBUF):
                pltpu.make_async_copy(
                    h_hbm.at[pl.ds(0, _C)], rows[b], gsem[b]).wait()
                pltpu.async_copy(
                    rows[b],
                    z_sh.at[idst_v.at[slot * _NBUF + b, pl.ds(0, _C)]],
                    ssem[b], add=True)
                if with_deg:
                    pltpu.async_copy(
                        ones_v,
                        deg_sh.at[idst_v.at[slot * _NBUF + b, pl.ds(0, _C)]],
                        dsem[b], add=True)
            iwait(isem[nslot])
            for b in range(_NBUF):
                pltpu.make_async_copy(
                    rows[b], z_sh.at[pl.ds(0, _C)], ssem[b]).wait()
                if with_deg:
                    pltpu.make_async_copy(
                        ones_v, deg_sh.at[pl.ds(0, _C)], dsem[b]).wait()
                pltpu.async_copy(
                    h_hbm.at[isrc_v.at[nslot * _NBUF + b, pl.ds(0, _C)]],
                    rows[b], gsem[b])
            nn = jnp.where(g + 2 < _NGRP, g + 2, 0)
            ifetch(nn, slot)

        def supergroup(G, carry):
            sub(2 * G, 0)
            sub(2 * G + 1, 1)
            return carry

        lax.fori_loop(0, _NGRP // 2, supergroup, 0)
        for b in range(_NBUF):
            pltpu.make_async_copy(
                h_hbm.at[pl.ds(0, _C)], rows[b], gsem[b]).wait()
        iwait(isem[1])
        iwait(jsem[0])
        iwait(jsem[1])
        plsc.subcore_barrier()
        pltpu.sync_copy(z_sh.at[pl.ds(s * _RPS, _RPS)],
                        z_out.at[c, pl.ds(s * _RPS, _RPS)])
        if with_deg:
            pltpu.sync_copy(deg_sh.at[pl.ds(s * _RPS, _RPS)],
                            deg_out.at[c, pl.ds(s * _RPS, _RPS)])

    return pl.kernel(body, out_type=out_type, mesh=_sc_mesh,
                     scratch_types=scratch)


_sc_agg_deg = _make_sc_agg(True)
_sc_agg = _make_sc_agg(False)


def _tc_body(h_ref, z_ref, deg_ref, ws_ref, wn_ref, b_ref, o_ref, *, relu):
    degsum = deg_ref[:, 0:1] + deg_ref[:, 1:2]
    inv = 1.0 / jnp.maximum(degsum, 1.0)
    hn = (z_ref[0] + z_ref[1]) * inv
    acc = jnp.dot(h_ref[...], ws_ref[...], preferred_element_type=jnp.float32)
    acc = acc + jnp.dot(hn, wn_ref[...], preferred_element_type=jnp.float32)
    acc = acc + b_ref[...]
    o_ref[...] = jnp.maximum(acc, 0.0) if relu else acc


def _tc_final_body(h_ref, z_ref, deg_ref, ws_ref, wn_ref, b_ref,
                   wo_ref, bo_ref, o_ref):
    degsum = deg_ref[:, 0:1] + deg_ref[:, 1:2]
    inv = 1.0 / jnp.maximum(degsum, 1.0)
    hn = (z_ref[0] + z_ref[1]) * inv
    acc = jnp.dot(h_ref[...], ws_ref[...], preferred_element_type=jnp.float32)
    acc = acc + jnp.dot(hn, wn_ref[...], preferred_element_type=jnp.float32)
    acc = acc + b_ref[...]
    out = jnp.dot(acc, wo_ref[...], preferred_element_type=jnp.float32)
    o_ref[...] = out + bo_ref[...]


_BM = 1024
_GRID = _NP // _BM

_layer_specs = [
    pl.BlockSpec((_BM, _D), lambda i: (i, 0)),
    pl.BlockSpec((2, _BM, _D), lambda i: (0, i, 0)),
    pl.BlockSpec((_BM, 2), lambda i: (i, 0)),
    pl.BlockSpec((_D, _D), lambda i: (0, 0)),
    pl.BlockSpec((_D, _D), lambda i: (0, 0)),
    pl.BlockSpec((1, _D), lambda i: (0, 0)),
]


def _tc_layer(h, z, degt, ws, wn, b, relu):
    return pl.pallas_call(
        functools.partial(_tc_body, relu=relu),
        out_shape=jax.ShapeDtypeStruct((_NP, _D), jnp.float32),
        grid=(_GRID,),
        in_specs=_layer_specs,
        out_specs=pl.BlockSpec((_BM, _D), lambda i: (i, 0)),
    )(h, z, degt, ws, wn, b.reshape(1, _D))


def _tc_final(h, z, degt, ws, wn, b, wo, bo):
    return pl.pallas_call(
        _tc_final_body,
        out_shape=jax.ShapeDtypeStruct((_NP, _D), jnp.float32),
        grid=(_GRID,),
        in_specs=_layer_specs + [
            pl.BlockSpec((_D, _D), lambda i: (0, 0)),
            pl.BlockSpec((1, _D), lambda i: (0, 0)),
        ],
        out_specs=pl.BlockSpec((_BM, _D), lambda i: (i, 0)),
    )(h, z, degt, ws, wn, b.reshape(1, _D), wo, bo.reshape(1, _D))


def kernel(features, edge_index, nonzer_index, nonzer_value,
           W_self1, W_neigh1, b1,
           W_self2, W_neigh2, b2,
           W_self3, W_neigh3, b3,
           W_out, b_out):
    del nonzer_index, nonzer_value
    x = jnp.pad(features, ((0, _NP - _N), (0, 0)))
    npad = _EPAD - _E
    srcp = jnp.pad(
        jnp.concatenate([edge_index[0], jnp.zeros((npad,), jnp.int32)])
        .reshape(_NW * _CPT, _C),
        ((0, 0), (0, 128 - _C))).reshape(_NW * _NGRP, _NBUF, 128)
    dstp = jnp.pad(
        jnp.concatenate([edge_index[1], jnp.full((npad,), _N, jnp.int32)])
        .reshape(_NW * _CPT, _C),
        ((0, 0), (0, 128 - _C)),
        constant_values=_N).reshape(_NW * _NGRP, _NBUF, 128)
    zrow = jnp.zeros((_NACC, _D), jnp.float32)
    zvec = jnp.zeros((_NACC,), jnp.float32)

    z, deg = _sc_agg_deg(x, srcp, dstp, zrow, zvec)
    degt = deg.T
    h = _tc_layer(x, z, degt, W_self1, W_neigh1, b1, True)
    z = _sc_agg(h, srcp, dstp, zrow)
    h = _tc_layer(h, z, degt, W_self2, W_neigh2, b2, True)
    z = _sc_agg(h, srcp, dstp, zrow)
    out = _tc_final(h, z, degt, W_self3, W_neigh3, b3, W_out, b_out)
    return out[:_N]


# speedup vs baseline: 1.7543x; 1.7543x over previous
import functools

import jax
import jax.numpy as jnp
from jax import lax
from jax.experimental import pallas as pl
from jax.experimental.pallas import tpu as pltpu
from jax.experimental.pallas import tpu_sc as plsc

_N = 10000
_NP = 10240
_NACC = 10240
_D = 128
_E = 320000
_C = 120
_CPT = 84
_NW = 32
_EPAD = _C * _CPT * _NW
_RPS = _NACC // 16
_NBUF = 3
_NGRP = _CPT // _NBUF


_sc_mesh = plsc.VectorSubcoreMesh(core_axis_name="c", subcore_axis_name="s")


def _make_sc_agg(with_deg):
    scratch = [
        pltpu.VMEM((2 * _NBUF, 128), jnp.int32),
        pltpu.VMEM((2 * _NBUF, 128), jnp.int32),
    ]
    scratch += [pltpu.VMEM((_C, _D), jnp.float32) for _ in range(_NBUF)]
    if with_deg:
        scratch += [pltpu.VMEM((128,), jnp.float32)]
    scratch += [pltpu.VMEM_SHARED((_NACC, _D), jnp.float32)]
    if with_deg:
        scratch += [pltpu.VMEM_SHARED((_NACC,), jnp.float32)]
    nsem = 2 * _NBUF + 4 + (_NBUF if with_deg else 0)
    scratch += [pltpu.SemaphoreType.DMA for _ in range(nsem)]
    if with_deg:
        out_type = (
            jax.ShapeDtypeStruct((2, _NP, _D), jnp.float32),
            jax.ShapeDtypeStruct((2, _NP), jnp.float32),
        )
    else:
        out_type = jax.ShapeDtypeStruct((2, _NP, _D), jnp.float32)

    def body(h_hbm, src_hbm, dst_hbm, zrow_hbm, *rest):
        if with_deg:
            zvec_hbm, z_out, deg_out, isrc_v, idst_v = rest[:5]
            bufs = rest[5:]
        else:
            z_out, isrc_v, idst_v = rest[:3]
            bufs = rest[3:]
        rows = bufs[:_NBUF]
        k = _NBUF
        if with_deg:
            ones_v = bufs[k]
            k += 1
        z_sh = bufs[k]
        k += 1
        if with_deg:
            deg_sh = bufs[k]
            k += 1
        sems = bufs[k:]
        gsem = sems[:_NBUF]
        ssem = sems[_NBUF:2 * _NBUF]
        off = 2 * _NBUF
        if with_deg:
            dsem = sems[off:off + _NBUF]
            off += _NBUF
        isem = sems[off:off + 2]
        jsem = sems[off + 2:off + 4]

        c = lax.axis_index("c")
        s = lax.axis_index("s")
        base = (c * 16 + s) * _NGRP
        pltpu.sync_copy(zrow_hbm.at[pl.ds(s * _RPS, _RPS)],
                        z_sh.at[pl.ds(s * _RPS, _RPS)])
        if with_deg:
            pltpu.sync_copy(zvec_hbm.at[pl.ds(s * _RPS, _RPS)],
                            deg_sh.at[pl.ds(s * _RPS, _RPS)])
            for i in range(128 // 16):
                ones_v[pl.ds(i * 16, 16)] = jnp.full((16,), 1.0, jnp.float32)
        plsc.subcore_barrier()

        def ifetch(g, slot):
            pltpu.async_copy(src_hbm.at[base + g],
                             isrc_v.at[pl.ds(slot * _NBUF, _NBUF)],
                             isem[slot])
            pltpu.async_copy(dst_hbm.at[base + g],
                             idst_v.at[pl.ds(slot * _NBUF, _NBUF)],
                             jsem[slot])

        def iwait(sem):
            pltpu.make_async_copy(src_hbm.at[0],
                                  isrc_v.at[pl.ds(0, _NBUF)], sem).wait()

        ifetch(0, 0)
        ifetch(1, 1)
        iwait(isem[0])
        for b in range(_NBUF):
            pltpu.async_copy(h_hbm.at[isrc_v.at[b, pl.ds(0, _C)]],
                             rows[b], gsem[b])

        def sub(g, slot):
            nslot = 1 - slot
            iwait(jsem[slot])
            for b in range(_NBUF):
                pltpu.make_async_copy(
                    h_hbm.at[pl.ds(0, _C)], rows[b], gsem[b]).wait()
                pltpu.async_copy(
                    rows[b],
                    z_sh.at[idst_v.at[slot * _NBUF + b, pl.ds(0, _C)]],
                    ssem[b], add=True)
                if with_deg:
                    pltpu.async_copy(
                        ones_v.at[pl.ds(0, _C)],
                        deg_sh.at[idst_v.at[slot * _NBUF + b, pl.ds(0, _C)]],
                        dsem[b], add=True)
            iwait(isem[nslot])
            for b in range(_NBUF):
                pltpu.make_async_copy(
                    rows[b], z_sh.at[pl.ds(0, _C)], ssem[b]).wait()
                if with_deg:
                    pltpu.make_async_copy(
                        ones_v.at[pl.ds(0, _C)], deg_sh.at[pl.ds(0, _C)],
                        dsem[b]).wait()
                pltpu.async_copy(
                    h_hbm.at[isrc_v.at[nslot * _NBUF + b, pl.ds(0, _C)]],
                    rows[b], gsem[b])
            nn = jnp.where(g + 2 < _NGRP, g + 2, 0)
            ifetch(nn, slot)

        def supergroup(G, carry):
            sub(2 * G, 0)
            sub(2 * G + 1, 1)
            return carry

        lax.fori_loop(0, _NGRP // 2, supergroup, 0)
        for b in range(_NBUF):
            pltpu.make_async_copy(
                h_hbm.at[pl.ds(0, _C)], rows[b], gsem[b]).wait()
        iwait(isem[1])
        iwait(jsem[0])
        iwait(jsem[1])
        plsc.subcore_barrier()
        pltpu.sync_copy(z_sh.at[pl.ds(s * _RPS, _RPS)],
                        z_out.at[c, pl.ds(s * _RPS, _RPS)])
        if with_deg:
            pltpu.sync_copy(deg_sh.at[pl.ds(s * _RPS, _RPS)],
                            deg_out.at[c, pl.ds(s * _RPS, _RPS)])

    return pl.kernel(body, out_type=out_type, mesh=_sc_mesh,
                     scratch_types=scratch)


_sc_agg_deg = _make_sc_agg(True)
_sc_agg = _make_sc_agg(False)


def _tc_body(h_ref, z_ref, deg_ref, ws_ref, wn_ref, b_ref, o_ref, *, relu):
    degsum = deg_ref[:, 0:1] + deg_ref[:, 1:2]
    inv = 1.0 / jnp.maximum(degsum, 1.0)
    hn = (z_ref[0] + z_ref[1]) * inv
    acc = jnp.dot(h_ref[...], ws_ref[...], preferred_element_type=jnp.float32)
    acc = acc + jnp.dot(hn, wn_ref[...], preferred_element_type=jnp.float32)
    acc = acc + b_ref[...]
    o_ref[...] = jnp.maximum(acc, 0.0) if relu else acc


def _tc_final_body(h_ref, z_ref, deg_ref, ws_ref, wn_ref, b_ref,
                   wo_ref, bo_ref, o_ref):
    degsum = deg_ref[:, 0:1] + deg_ref[:, 1:2]
    inv = 1.0 / jnp.maximum(degsum, 1.0)
    hn = (z_ref[0] + z_ref[1]) * inv
    acc = jnp.dot(h_ref[...], ws_ref[...], preferred_element_type=jnp.float32)
    acc = acc + jnp.dot(hn, wn_ref[...], preferred_element_type=jnp.float32)
    acc = acc + b_ref[...]
    out = jnp.dot(acc, wo_ref[...], preferred_element_type=jnp.float32)
    o_ref[...] = out + bo_ref[...]


_BM = 1024
_GRID = _NP // _BM

_layer_specs = [
    pl.BlockSpec((_BM, _D), lambda i: (i, 0)),
    pl.BlockSpec((2, _BM, _D), lambda i: (0, i, 0)),
    pl.BlockSpec((_BM, 2), lambda i: (i, 0)),
    pl.BlockSpec((_D, _D), lambda i: (0, 0)),
    pl.BlockSpec((_D, _D), lambda i: (0, 0)),
    pl.BlockSpec((1, _D), lambda i: (0, 0)),
]


def _tc_layer(h, z, degt, ws, wn, b, relu):
    return pl.pallas_call(
        functools.partial(_tc_body, relu=relu),
        out_shape=jax.ShapeDtypeStruct((_NP, _D), jnp.float32),
        grid=(_GRID,),
        in_specs=_layer_specs,
        out_specs=pl.BlockSpec((_BM, _D), lambda i: (i, 0)),
    )(h, z, degt, ws, wn, b.reshape(1, _D))


def _tc_final(h, z, degt, ws, wn, b, wo, bo):
    return pl.pallas_call(
        _tc_final_body,
        out_shape=jax.ShapeDtypeStruct((_NP, _D), jnp.float32),
        grid=(_GRID,),
        in_specs=_layer_specs + [
            pl.BlockSpec((_D, _D), lambda i: (0, 0)),
            pl.BlockSpec((1, _D), lambda i: (0, 0)),
        ],
        out_specs=pl.BlockSpec((_BM, _D), lambda i: (i, 0)),
    )(h, z, degt, ws, wn, b.reshape(1, _D), wo, bo.reshape(1, _D))


def kernel(features, edge_index, nonzer_index, nonzer_value,
           W_self1, W_neigh1, b1,
           W_self2, W_neigh2, b2,
           W_self3, W_neigh3, b3,
           W_out, b_out):
    del nonzer_index, nonzer_value
    x = jnp.pad(features, ((0, _NP - _N), (0, 0)))
    npad = _EPAD - _E
    srcp = jnp.pad(
        jnp.concatenate([edge_index[0], jnp.zeros((npad,), jnp.int32)])
        .reshape(_NW * _CPT, _C),
        ((0, 0), (0, 128 - _C))).reshape(_NW * _NGRP, _NBUF, 128)
    dstp = jnp.pad(
        jnp.concatenate([edge_index[1], jnp.full((npad,), _N, jnp.int32)])
        .reshape(_NW * _CPT, _C),
        ((0, 0), (0, 128 - _C)),
        constant_values=_N).reshape(_NW * _NGRP, _NBUF, 128)
    zrow = jnp.zeros((_NACC, _D), jnp.float32)
    zvec = jnp.zeros((_NACC,), jnp.float32)

    z, deg = _sc_agg_deg(x, srcp, dstp, zrow, zvec)
    degt = deg.T
    h = _tc_layer(x, z, degt, W_self1, W_neigh1, b1, True)
    z = _sc_agg(h, srcp, dstp, zrow)
    h = _tc_layer(h, z, degt, W_self2, W_neigh2, b2, True)
    z = _sc_agg(h, srcp, dstp, zrow)
    out = _tc_final(h, z, degt, W_self3, W_neigh3, b3, W_out, b_out)
    return out[:_N]


# speedup vs baseline: 1.8953x; 1.0804x over previous
import functools

import jax
import jax.numpy as jnp
from jax import lax
from jax.experimental import pallas as pl
from jax.experimental.pallas import tpu as pltpu
from jax.experimental.pallas import tpu_sc as plsc

_N = 10000
_NP = 10240
_NACC = 10240
_D = 128
_E = 320000
_C = 112
_CPT = 90
_NW = 32
_EPAD = _C * _CPT * _NW
_RPS = _NACC // 16
_NBUF = 3
_NGRP = _CPT // _NBUF


_sc_mesh = plsc.VectorSubcoreMesh(core_axis_name="c", subcore_axis_name="s")


def _make_sc_agg(with_deg):
    scratch = [
        pltpu.VMEM((2 * _NBUF, 128), jnp.int32),
        pltpu.VMEM((2 * _NBUF, 128), jnp.int32),
    ]
    scratch += [pltpu.VMEM((_C, _D), jnp.float32) for _ in range(_NBUF)]
    if with_deg:
        scratch += [pltpu.VMEM((128,), jnp.float32)]
    scratch += [pltpu.VMEM_SHARED((_NACC, _D), jnp.float32)]
    if with_deg:
        scratch += [pltpu.VMEM_SHARED((_NACC,), jnp.float32)]
    nsem = 2 * _NBUF + 4 + (_NBUF if with_deg else 0)
    scratch += [pltpu.SemaphoreType.DMA for _ in range(nsem)]
    if with_deg:
        out_type = (
            jax.ShapeDtypeStruct((2, _NP, _D), jnp.float32),
            jax.ShapeDtypeStruct((2, _NP), jnp.float32),
        )
    else:
        out_type = jax.ShapeDtypeStruct((2, _NP, _D), jnp.float32)

    def body(h_hbm, src_hbm, dst_hbm, zrow_hbm, *rest):
        if with_deg:
            zvec_hbm, z_out, deg_out, isrc_v, idst_v = rest[:5]
            bufs = rest[5:]
        else:
            z_out, isrc_v, idst_v = rest[:3]
            bufs = rest[3:]
        rows = bufs[:_NBUF]
        k = _NBUF
        if with_deg:
            ones_v = bufs[k]
            k += 1
        z_sh = bufs[k]
        k += 1
        if with_deg:
            deg_sh = bufs[k]
            k += 1
        sems = bufs[k:]
        gsem = sems[:_NBUF]
        ssem = sems[_NBUF:2 * _NBUF]
        off = 2 * _NBUF
        if with_deg:
            dsem = sems[off:off + _NBUF]
            off += _NBUF
        isem = sems[off:off + 2]
        jsem = sems[off + 2:off + 4]

        c = lax.axis_index("c")
        s = lax.axis_index("s")
        base = (c * 16 + s) * _NGRP
        pltpu.sync_copy(zrow_hbm.at[pl.ds(s * _RPS, _RPS)],
                        z_sh.at[pl.ds(s * _RPS, _RPS)])
        if with_deg:
            pltpu.sync_copy(zvec_hbm.at[pl.ds(s * _RPS, _RPS)],
                            deg_sh.at[pl.ds(s * _RPS, _RPS)])
            for i in range(128 // 16):
                ones_v[pl.ds(i * 16, 16)] = jnp.full((16,), 1.0, jnp.float32)
        plsc.subcore_barrier()

        def ifetch(g, slot):
            pltpu.async_copy(src_hbm.at[base + g],
                             isrc_v.at[pl.ds(slot * _NBUF, _NBUF)],
                             isem[slot])
            pltpu.async_copy(dst_hbm.at[base + g],
                             idst_v.at[pl.ds(slot * _NBUF, _NBUF)],
                             jsem[slot])

        def iwait(sem):
            pltpu.make_async_copy(src_hbm.at[0],
                                  isrc_v.at[pl.ds(0, _NBUF)], sem).wait()

        ifetch(0, 0)
        ifetch(1, 1)
        iwait(isem[0])
        for b in range(_NBUF):
            pltpu.async_copy(h_hbm.at[isrc_v.at[b, pl.ds(0, _C)]],
                             rows[b], gsem[b])

        def sub(g, slot):
            nslot = 1 - slot
            iwait(jsem[slot])
            for b in range(_NBUF):
                pltpu.make_async_copy(
                    h_hbm.at[pl.ds(0, _C)], rows[b], gsem[b]).wait()
                pltpu.async_copy(
                    rows[b],
                    z_sh.at[idst_v.at[slot * _NBUF + b, pl.ds(0, _C)]],
                    ssem[b], add=True)
                if with_deg:
                    pltpu.async_copy(
                        ones_v.at[pl.ds(0, _C)],
                        deg_sh.at[idst_v.at[slot * _NBUF + b, pl.ds(0, _C)]],
                        dsem[b], add=True)
            iwait(isem[nslot])
            for b in range(_NBUF):
                pltpu.make_async_copy(
                    rows[b], z_sh.at[pl.ds(0, _C)], ssem[b]).wait()
                if with_deg:
                    pltpu.make_async_copy(
                        ones_v.at[pl.ds(0, _C)], deg_sh.at[pl.ds(0, _C)],
                        dsem[b]).wait()
                pltpu.async_copy(
                    h_hbm.at[isrc_v.at[nslot * _NBUF + b, pl.ds(0, _C)]],
                    rows[b], gsem[b])
            nn = jnp.where(g + 2 < _NGRP, g + 2, 0)
            ifetch(nn, slot)

        def supergroup(G, carry):
            sub(2 * G, 0)
            sub(2 * G + 1, 1)
            return carry

        lax.fori_loop(0, _NGRP // 2, supergroup, 0)
        for b in range(_NBUF):
            pltpu.make_async_copy(
                h_hbm.at[pl.ds(0, _C)], rows[b], gsem[b]).wait()
        iwait(isem[1])
        iwait(jsem[0])
        iwait(jsem[1])
        plsc.subcore_barrier()
        pltpu.sync_copy(z_sh.at[pl.ds(s * _RPS, _RPS)],
                        z_out.at[c, pl.ds(s * _RPS, _RPS)])
        if with_deg:
            pltpu.sync_copy(deg_sh.at[pl.ds(s * _RPS, _RPS)],
                            deg_out.at[c, pl.ds(s * _RPS, _RPS)])

    return pl.kernel(body, out_type=out_type, mesh=_sc_mesh,
                     scratch_types=scratch)


_sc_agg_deg = _make_sc_agg(True)
_sc_agg = _make_sc_agg(False)


def _tc_body(h_ref, z_ref, deg_ref, ws_ref, wn_ref, b_ref, o_ref, *, relu):
    degsum = deg_ref[:, 0:1] + deg_ref[:, 1:2]
    inv = 1.0 / jnp.maximum(degsum, 1.0)
    hn = (z_ref[0] + z_ref[1]) * inv
    acc = jnp.dot(h_ref[...], ws_ref[...], preferred_element_type=jnp.float32)
    acc = acc + jnp.dot(hn, wn_ref[...], preferred_element_type=jnp.float32)
    acc = acc + b_ref[...]
    o_ref[...] = jnp.maximum(acc, 0.0) if relu else acc


def _tc_final_body(h_ref, z_ref, deg_ref, ws_ref, wn_ref, b_ref,
                   wo_ref, bo_ref, o_ref):
    degsum = deg_ref[:, 0:1] + deg_ref[:, 1:2]
    inv = 1.0 / jnp.maximum(degsum, 1.0)
    hn = (z_ref[0] + z_ref[1]) * inv
    acc = jnp.dot(h_ref[...], ws_ref[...], preferred_element_type=jnp.float32)
    acc = acc + jnp.dot(hn, wn_ref[...], preferred_element_type=jnp.float32)
    acc = acc + b_ref[...]
    out = jnp.dot(acc, wo_ref[...], preferred_element_type=jnp.float32)
    o_ref[...] = out + bo_ref[...]


_BM = 1024
_GRID = _NP // _BM

_layer_specs = [
    pl.BlockSpec((_BM, _D), lambda i: (i, 0)),
    pl.BlockSpec((2, _BM, _D), lambda i: (0, i, 0)),
    pl.BlockSpec((_BM, 2), lambda i: (i, 0)),
    pl.BlockSpec((_D, _D), lambda i: (0, 0)),
    pl.BlockSpec((_D, _D), lambda i: (0, 0)),
    pl.BlockSpec((1, _D), lambda i: (0, 0)),
]


def _tc_layer(h, z, degt, ws, wn, b, relu):
    return pl.pallas_call(
        functools.partial(_tc_body, relu=relu),
        out_shape=jax.ShapeDtypeStruct((_NP, _D), jnp.float32),
        grid=(_GRID,),
        in_specs=_layer_specs,
        out_specs=pl.BlockSpec((_BM, _D), lambda i: (i, 0)),
    )(h, z, degt, ws, wn, b.reshape(1, _D))


def _tc_final(h, z, degt, ws, wn, b, wo, bo):
    return pl.pallas_call(
        _tc_final_body,
        out_shape=jax.ShapeDtypeStruct((_NP, _D), jnp.float32),
        grid=(_GRID,),
        in_specs=_layer_specs + [
            pl.BlockSpec((_D, _D), lambda i: (0, 0)),
            pl.BlockSpec((1, _D), lambda i: (0, 0)),
        ],
        out_specs=pl.BlockSpec((_BM, _D), lambda i: (i, 0)),
    )(h, z, degt, ws, wn, b.reshape(1, _D), wo, bo.reshape(1, _D))


def kernel(features, edge_index, nonzer_index, nonzer_value,
           W_self1, W_neigh1, b1,
           W_self2, W_neigh2, b2,
           W_self3, W_neigh3, b3,
           W_out, b_out):
    del nonzer_index, nonzer_value
    x = jnp.pad(features, ((0, _NP - _N), (0, 0)))
    npad = _EPAD - _E
    srcp = jnp.pad(
        jnp.concatenate([edge_index[0], jnp.zeros((npad,), jnp.int32)])
        .reshape(_NW * _CPT, _C),
        ((0, 0), (0, 128 - _C))).reshape(_NW * _NGRP, _NBUF, 128)
    dstp = jnp.pad(
        jnp.concatenate([edge_index[1], jnp.full((npad,), _N, jnp.int32)])
        .reshape(_NW * _CPT, _C),
        ((0, 0), (0, 128 - _C)),
        constant_values=_N).reshape(_NW * _NGRP, _NBUF, 128)
    zrow = jnp.zeros((_NACC, _D), jnp.float32)
    zvec = jnp.zeros((_NACC,), jnp.float32)

    z, deg = _sc_agg_deg(x, srcp, dstp, zrow, zvec)
    degt = deg.T
    h = _tc_layer(x, z, degt, W_self1, W_neigh1, b1, True)
    z = _sc_agg(h, srcp, dstp, zrow)
    h = _tc_layer(h, z, degt, W_self2, W_neigh2, b2, True)
    z = _sc_agg(h, srcp, dstp, zrow)
    out = _tc_final(h, z, degt, W_self3, W_neigh3, b3, W_out, b_out)
    return out[:_N]
